# Initial kernel scaffold; baseline (speedup 1.0000x reference)
import jax
import jax.numpy as jnp
from jax.experimental import pallas as pl


def kernel(x, w1, b1, w2, b2, w3, b3, w4, b4):
    raise NotImplementedError("write your pallas kernel here")


# speedup vs baseline: 1.2104x; 1.2104x over previous
import functools

import jax
import jax.numpy as jnp
from jax.experimental import pallas as pl
from jax.experimental.pallas import tpu as pltpu


def _round_up(n, m):
    return (n + m - 1) // m * m


def _mlp_kernel(x_ref, w1_ref, ws_ref, b_ref, out_ref, *, out_features):
    h = jnp.dot(x_ref[...], w1_ref[...],
                preferred_element_type=jnp.float32) + b_ref[0]
    h = jnp.maximum(h, 0.0)
    for i in range(3):
        h = jnp.dot(h, ws_ref[i], preferred_element_type=jnp.float32) + b_ref[i + 1]
        if i < 2:
            h = jnp.maximum(h, 0.0)
    out_ref[...] = h[:, :out_features]


def kernel(x, w1, b1, w2, b2, w3, b3, w4, b4):
    B, F = x.shape
    out_features = w4.shape[1]
    H = 128

    w1p = jnp.zeros((F, H), jnp.float32).at[:, :w1.shape[1]].set(w1)
    ws = jnp.zeros((3, H, H), jnp.float32)
    for i, w in enumerate((w2, w3, w4)):
        ws = ws.at[i, :w.shape[0], :w.shape[1]].set(w)
    bs = jnp.zeros((4, 1, H), jnp.float32)
    for i, b in enumerate((b1, b2, b3, b4)):
        bs = bs.at[i, :, :b.shape[1]].set(b)

    TB = min(512, _round_up(B, 8))
    B_pad = _round_up(B, TB)
    if B_pad != B:
        x = jnp.zeros((B_pad, F), x.dtype).at[:B].set(x)

    grid = (B_pad // TB,)
    flops = 2 * B_pad * (F * H + 3 * H * H)
    bytes_accessed = 4 * (B_pad * F + F * H + 3 * H * H + 4 * H
                          + B_pad * out_features)

    out = pl.pallas_call(
        functools.partial(_mlp_kernel, out_features=out_features),
        out_shape=jax.ShapeDtypeStruct((B_pad, out_features), jnp.float32),
        grid=grid,
        in_specs=[
            pl.BlockSpec((TB, F), lambda i: (i, 0)),
            pl.BlockSpec((F, H), lambda i: (0, 0)),
            pl.BlockSpec((3, H, H), lambda i: (0, 0, 0)),
            pl.BlockSpec((4, 1, H), lambda i: (0, 0, 0)),
        ],
        out_specs=pl.BlockSpec((TB, out_features), lambda i: (i, 0)),
        compiler_params=pltpu.CompilerParams(
            dimension_semantics=("parallel",),
            vmem_limit_bytes=64 * 1024 * 1024,
        ),
        cost_estimate=pl.CostEstimate(
            flops=flops, transcendentals=0, bytes_accessed=bytes_accessed),
    )(x, w1p, ws, bs)

    return out[:B]


# speedup vs baseline: 1.2118x; 1.0012x over previous
import functools

import jax
import jax.numpy as jnp
from jax.experimental import pallas as pl
from jax.experimental.pallas import tpu as pltpu


def _round_up(n, m):
    return (n + m - 1) // m * m


def _mlp_kernel(x_ref, w1_ref, ws_ref, b_ref, out_ref, *, out_features):
    h = jnp.dot(x_ref[...], w1_ref[...],
                preferred_element_type=jnp.float32) + b_ref[0]
    h = jnp.maximum(h, 0.0)
    for i in range(3):
        h = jnp.dot(h, ws_ref[i], preferred_element_type=jnp.float32) + b_ref[i + 1]
        if i < 2:
            h = jnp.maximum(h, 0.0)
    out_ref[...] = h


def kernel(x, w1, b1, w2, b2, w3, b3, w4, b4):
    B, F = x.shape
    out_features = w4.shape[1]
    H = 128

    w1p = jnp.zeros((F, H), jnp.float32).at[:, :w1.shape[1]].set(w1)
    ws = jnp.zeros((3, H, H), jnp.float32)
    for i, w in enumerate((w2, w3, w4)):
        ws = ws.at[i, :w.shape[0], :w.shape[1]].set(w)
    bs = jnp.zeros((4, 1, H), jnp.float32)
    for i, b in enumerate((b1, b2, b3, b4)):
        bs = bs.at[i, :, :b.shape[1]].set(b)

    TB = min(512, _round_up(B, 8))
    B_pad = _round_up(B, TB)
    if B_pad != B:
        x = jnp.zeros((B_pad, F), x.dtype).at[:B].set(x)

    grid = (B_pad // TB,)
    flops = 2 * B_pad * (F * H + 3 * H * H)
    bytes_accessed = 4 * (B_pad * F + F * H + 3 * H * H + 4 * H
                          + B_pad * out_features)

    out = pl.pallas_call(
        functools.partial(_mlp_kernel, out_features=out_features),
        out_shape=jax.ShapeDtypeStruct((B_pad, H), jnp.float32),
        grid=grid,
        in_specs=[
            pl.BlockSpec((TB, F), lambda i: (i, 0)),
            pl.BlockSpec((F, H), lambda i: (0, 0)),
            pl.BlockSpec((3, H, H), lambda i: (0, 0, 0)),
            pl.BlockSpec((4, 1, H), lambda i: (0, 0, 0)),
        ],
        out_specs=pl.BlockSpec((TB, H), lambda i: (i, 0)),
        compiler_params=pltpu.CompilerParams(
            dimension_semantics=("parallel",),
            vmem_limit_bytes=64 * 1024 * 1024,
        ),
        cost_estimate=pl.CostEstimate(
            flops=flops, transcendentals=0, bytes_accessed=bytes_accessed),
    )(x, w1p, ws, bs)

    return out[:B, :out_features]


# speedup vs baseline: 1.5097x; 1.2458x over previous
import jax
import jax.numpy as jnp
from jax.experimental import pallas as pl
from jax.experimental.pallas import tpu as pltpu


def _round_up(n, m):
    return (n + m - 1) // m * m


def _mlp_kernel(x_ref, w1_ref, b1_ref, w2_ref, b2_ref, w3_ref, b3_ref,
                w4_ref, b4_ref, out_ref):
    h = jnp.dot(x_ref[...], w1_ref[...],
                preferred_element_type=jnp.float32) + b1_ref[...]
    h = jnp.maximum(h, 0.0)
    h = jnp.dot(h, w2_ref[...], preferred_element_type=jnp.float32) + b2_ref[...]
    h = jnp.maximum(h, 0.0)
    h = jnp.dot(h, w3_ref[...], preferred_element_type=jnp.float32) + b3_ref[...]
    h = jnp.maximum(h, 0.0)
    out_ref[...] = jnp.dot(h, w4_ref[...],
                           preferred_element_type=jnp.float32) + b4_ref[...]


def kernel(x, w1, b1, w2, b2, w3, b3, w4, b4):
    B, F = x.shape
    out_features = w4.shape[1]

    TB = min(512, _round_up(B, 8))
    B_pad = _round_up(B, TB)
    if B_pad != B:
        x = jnp.zeros((B_pad, F), x.dtype).at[:B].set(x)

    grid = (B_pad // TB,)
    flops = 2 * B_pad * (F * w1.shape[1] + w2.size + w3.size + w4.size)
    bytes_accessed = 4 * (B_pad * F + w1.size + w2.size + w3.size + w4.size
                          + B_pad * out_features)

    whole = lambda shape: pl.BlockSpec(shape, lambda i: tuple(0 for _ in shape))

    out = pl.pallas_call(
        _mlp_kernel,
        out_shape=jax.ShapeDtypeStruct((B_pad, out_features), jnp.float32),
        grid=grid,
        in_specs=[
            pl.BlockSpec((TB, F), lambda i: (i, 0)),
            whole(w1.shape), whole(b1.shape),
            whole(w2.shape), whole(b2.shape),
            whole(w3.shape), whole(b3.shape),
            whole(w4.shape), whole(b4.shape),
        ],
        out_specs=pl.BlockSpec((TB, out_features), lambda i: (i, 0)),
        compiler_params=pltpu.CompilerParams(
            dimension_semantics=("parallel",),
            vmem_limit_bytes=64 * 1024 * 1024,
        ),
        cost_estimate=pl.CostEstimate(
            flops=flops, transcendentals=0, bytes_accessed=bytes_accessed),
    )(x, w1, b1, w2, b2, w3, b3, w4, b4)

    return out[:B]


# speedup vs baseline: 2.3159x; 1.5340x over previous
import jax
import jax.numpy as jnp
from jax.experimental import pallas as pl
from jax.experimental.pallas import tpu as pltpu


def _round_up(n, m):
    return (n + m - 1) // m * m


def _mlp_kernel(x_ref, w1_ref, b1_ref, w2_ref, b2_ref, w3_ref, b3_ref,
                w4_ref, b4_ref, out_ref):
    h = jnp.dot(x_ref[...], w1_ref[...],
                preferred_element_type=jnp.float32) + b1_ref[...]
    h = jnp.maximum(h, 0.0)
    h = jnp.dot(h, w2_ref[...], preferred_element_type=jnp.float32) + b2_ref[...]
    h = jnp.maximum(h, 0.0)
    h = jnp.dot(h, w3_ref[...], preferred_element_type=jnp.float32) + b3_ref[...]
    h = jnp.maximum(h, 0.0)
    out_ref[...] = jnp.dot(h, w4_ref[...],
                           preferred_element_type=jnp.float32) + b4_ref[...]


def kernel(x, w1, b1, w2, b2, w3, b3, w4, b4):
    B, F = x.shape
    out_features = w4.shape[1]

    TB = min(2048, _round_up(B, 8))
    B_pad = _round_up(B, TB)
    if B_pad != B:
        x = jnp.zeros((B_pad, F), x.dtype).at[:B].set(x)

    grid = (B_pad // TB,)
    flops = 2 * B_pad * (F * w1.shape[1] + w2.size + w3.size + w4.size)
    bytes_accessed = 4 * (B_pad * F + w1.size + w2.size + w3.size + w4.size
                          + B_pad * out_features)

    whole = lambda shape: pl.BlockSpec(shape, lambda i: tuple(0 for _ in shape))

    out = pl.pallas_call(
        _mlp_kernel,
        out_shape=jax.ShapeDtypeStruct((B_pad, out_features), jnp.float32),
        grid=grid,
        in_specs=[
            pl.BlockSpec((TB, F), lambda i: (i, 0)),
            whole(w1.shape), whole(b1.shape),
            whole(w2.shape), whole(b2.shape),
            whole(w3.shape), whole(b3.shape),
            whole(w4.shape), whole(b4.shape),
        ],
        out_specs=pl.BlockSpec((TB, out_features), lambda i: (i, 0)),
        compiler_params=pltpu.CompilerParams(
            dimension_semantics=("parallel",),
            vmem_limit_bytes=64 * 1024 * 1024,
        ),
        cost_estimate=pl.CostEstimate(
            flops=flops, transcendentals=0, bytes_accessed=bytes_accessed),
    )(x, w1, b1, w2, b2, w3, b3, w4, b4)

    return out[:B]
